# baseline (speedup 1.0000x reference)
import functools

import jax
import jax.numpy as jnp
from jax import lax
from jax.experimental import pallas as pl
from jax.experimental.pallas import tpu as pltpu
from jax.experimental.pallas import tpu_sc as plsc


def _matmul_body(hx_ref, w_ref, o_ref):
    o_ref[...] = jnp.dot(hx_ref[...], w_ref[...],
                         preferred_element_type=jnp.float32,
                         precision=jax.lax.Precision.HIGHEST
                         ).astype(jnp.bfloat16)


def _addrelu_body(a_ref, b_ref, o_ref):
    o_ref[...] = jnp.maximum(a_ref[...] + b_ref[...], 0.0)


def _make_sc_spmm(N, E, GK, D, DP):
    info = plsc.get_sparse_core_info()
    NC, NS, L = info.num_cores, info.num_subcores, info.num_lanes
    NW = NC * NS
    B = 64
    TT = E // B
    TBASE = TT // NW
    TREM = TT % NW
    WB = 40
    SB = ((N + NS - 1) // NS + WB - 1) // WB * WB
    NBLK = SB // WB
    assert E % B == 0 and N % WB == 0
    mesh = plsc.VectorSubcoreMesh(core_axis_name="c", subcore_axis_name="s")

    @functools.partial(
        pl.kernel,
        out_type=jax.ShapeDtypeStruct((NC, N, D), jnp.float32),
        mesh=mesh,
        compiler_params=pltpu.CompilerParams(needs_layout_passes=False),
        scratch_types=[
            pltpu.VMEM((2 + GK, B), jnp.int32),
            pltpu.VMEM((2 + GK, B), jnp.int32),
            pltpu.VMEM((B, DP // 2), jnp.int32),
            pltpu.VMEM((B, DP // 2), jnp.int32),
            pltpu.VMEM((B, D), jnp.float32),
            pltpu.VMEM_SHARED((N, D), jnp.float32),
            pltpu.SemaphoreType.DMA,
            pltpu.SemaphoreType.DMA,
        ],
    )
    def sc_spmm(p_hbm, rec_hbm, out_hbm,
                rec0_v, rec1_v, rows0_v, rows1_v, msg_v, acc_sh,
                sem0, sem1):
        cid = lax.axis_index("c")
        sid = lax.axis_index("s")
        wid = sid * NC + cid
        rec_v = (rec0_v, rec1_v)
        rows_v = (rows0_v, rows1_v)
        sem = (sem0, sem1)

        def zero_row(r, carry):
            for dd in range(D // L):
                msg_v[r, pl.ds(dd * L, L)] = jnp.zeros((L,), jnp.float32)
            return carry
        lax.fori_loop(0, WB, zero_row, 0)
        row0 = sid * SB
        for b in range(NBLK):
            @pl.when(row0 + b * WB < N)
            def _():
                pltpu.sync_copy(msg_v.at[pl.ds(0, WB)],
                                acc_sh.at[pl.ds(row0 + b * WB, WB)])
        plsc.subcore_barrier()

        g0 = TBASE * wid + jnp.minimum(wid, TREM)
        tcnt = TBASE + (wid < TREM).astype(jnp.int32)

        def load(g, s):
            pltpu.sync_copy(rec_hbm.at[g], rec_v[s])
            pltpu.async_copy(p_hbm.at[rec_v[s].at[0]], rows_v[s], sem[s])

        def wait_gather(s):
            pltpu.make_async_copy(
                p_hbm.at[rec_v[s].at[0]], rows_v[s], sem[s]).wait()

        def compute_scatter(s):
            rv = rows_v[s]
            wv = rec_v[s]

            UNR = 4

            def _edges(jj, carry2):
                for u in range(UNR):
                    j = jj * UNR + u
                    wvec = [
                        plsc.bitcast(
                            plsc.load_gather(
                                wv, [jnp.full((L,), 2 + c, jnp.int32),
                                     jnp.full((L,), j, jnp.int32)]),
                            jnp.float32)
                        for c in range(GK)
                    ]
                    for dd in range(D // (2 * L)):
                        ma = None
                        mb = None
                        for c in range(GK):
                            words = rv[j, pl.ds(c * (D // 2) + dd * L, L)]
                            ab = plsc.bitcast(words, jnp.bfloat16)
                            a, b = plsc.unpack(
                                ab, format=plsc.PackFormat.INTERLEAVED,
                                preferred_element_type=jnp.float32)
                            ma = (wvec[c] * a if ma is None
                                  else ma + wvec[c] * a)
                            mb = (wvec[c] * b if mb is None
                                  else mb + wvec[c] * b)
                        msg_v[j, pl.ds(dd * 2 * L, L)] = ma
                        msg_v[j, pl.ds(dd * 2 * L + L, L)] = mb
                return carry2
            lax.fori_loop(0, B // UNR, _edges, 0)

            pltpu.sync_copy(msg_v, acc_sh.at[rec_v[s].at[1]], add=True)

        load(g0, 0)

        def pair_body(p, carry):
            g = g0 + 2 * p
            load(g + 1, 1)
            wait_gather(0)
            compute_scatter(0)

            @pl.when(2 * p + 2 < tcnt)
            def _():
                load(g + 2, 0)
            wait_gather(1)
            compute_scatter(1)
            return carry
        lax.fori_loop(0, tcnt // 2, pair_body, 0)

        @pl.when(tcnt % 2 == 1)
        def _():
            wait_gather(0)
            compute_scatter(0)

        plsc.subcore_barrier()
        for b in range(NBLK):
            @pl.when(row0 + b * WB < N)
            def _():
                r0 = row0 + b * WB
                pltpu.sync_copy(acc_sh.at[pl.ds(r0, WB)],
                                msg_v.at[pl.ds(0, WB)])
                pltpu.sync_copy(msg_v.at[pl.ds(0, WB)],
                                out_hbm.at[cid, pl.ds(r0, WB)])

    return sc_spmm, NC, B


def kernel(h, x, edge_weight, weights, inp_weights, graph_mixing_weight,
           inp_graph_mixing_weight, edge_index):
    N, D_IN = h.shape
    D_NET = x.shape[1]
    G, K, _, D_OUT = weights.shape
    E = edge_index.shape[1]
    GK = G * K
    DP = GK * D_OUT

    wh = weights * graph_mixing_weight[:, 0][:, None, None, None]
    wx = inp_weights * inp_graph_mixing_weight[:, 0][:, None, None, None]
    wh = wh.reshape(GK, D_IN, D_OUT).transpose(1, 0, 2).reshape(D_IN, DP)
    wx = wx.reshape(GK, D_NET, D_OUT).transpose(1, 0, 2).reshape(D_NET, DP)
    wcat = jnp.concatenate([wh, wx], axis=0)
    m = jnp.arange(D_OUT)
    jcol = 16 * (2 * (m // 32) + (m % 32) % 2) + (m % 32) // 2
    colperm = jnp.concatenate([c * D_OUT + jcol for c in range(GK)])
    wcat = wcat[:, colperm]
    hx = jnp.concatenate([h, x], axis=1)

    BLK = 1000
    p = pl.pallas_call(
        _matmul_body,
        grid=(N // BLK,),
        in_specs=[
            pl.BlockSpec((BLK, D_IN + D_NET), lambda i: (i, 0)),
            pl.BlockSpec((D_IN + D_NET, DP), lambda i: (0, 0)),
        ],
        out_specs=pl.BlockSpec((BLK, DP), lambda i: (i, 0)),
        out_shape=jax.ShapeDtypeStruct((N, DP), jnp.bfloat16),
    )(hx, wcat)

    sc_spmm, NC, B = _make_sc_spmm(N, E, GK, D_OUT, DP)
    TT = E // B
    eint = edge_index.reshape(2, TT, B).transpose(1, 0, 2)
    wbits = jax.lax.bitcast_convert_type(
        edge_weight.reshape(GK, TT, B), jnp.int32).transpose(1, 0, 2)
    rec = jnp.concatenate([eint, wbits], axis=1)
    p32 = jax.lax.bitcast_convert_type(
        p.reshape(N, DP // 2, 2), jnp.int32)
    parts = sc_spmm(p32, rec)

    out = pl.pallas_call(
        _addrelu_body,
        grid=(N // BLK,),
        in_specs=[
            pl.BlockSpec((BLK, D_OUT), lambda i: (i, 0)),
            pl.BlockSpec((BLK, D_OUT), lambda i: (i, 0)),
        ],
        out_specs=pl.BlockSpec((BLK, D_OUT), lambda i: (i, 0)),
        out_shape=jax.ShapeDtypeStruct((N, D_OUT), jnp.float32),
    )(parts[0], parts[1])
    return out


# speedup vs baseline: 1.0664x; 1.0664x over previous
import functools

import jax
import jax.numpy as jnp
from jax import lax
from jax.experimental import pallas as pl
from jax.experimental.pallas import tpu as pltpu
from jax.experimental.pallas import tpu_sc as plsc


def _matmul_body(hx_ref, w_ref, o_ref):
    o_ref[...] = jnp.dot(hx_ref[...], w_ref[...],
                         preferred_element_type=jnp.float32,
                         precision=jax.lax.Precision.HIGHEST
                         ).astype(jnp.bfloat16)


def _addrelu_body(a_ref, b_ref, o_ref):
    o_ref[...] = jnp.maximum(a_ref[...] + b_ref[...], 0.0)


def _make_sc_spmm(N, E, GK, D, DP):
    info = plsc.get_sparse_core_info()
    NC, NS, L = info.num_cores, info.num_subcores, info.num_lanes
    NW = NC * NS
    B = 64
    RB = 8
    TT = E // B
    CB = (TT // NW) // RB * RB
    CA = CB + RB
    XA = (TT - NW * CB) // RB
    WB = 40
    SB = ((N + NS - 1) // NS + WB - 1) // WB * WB
    NBLK = SB // WB
    assert E % B == 0 and N % WB == 0 and TT % RB == 0
    assert XA * CA + (NW - XA) * CB == TT and 0 <= XA <= NW
    assert CB // RB >= 2 and RB % 2 == 0
    mesh = plsc.VectorSubcoreMesh(core_axis_name="c", subcore_axis_name="s")

    @functools.partial(
        pl.kernel,
        out_type=jax.ShapeDtypeStruct((NC, N, D), jnp.float32),
        mesh=mesh,
        compiler_params=pltpu.CompilerParams(needs_layout_passes=False),
        scratch_types=[
            pltpu.VMEM((2, RB, 2, B), jnp.int32),
            pltpu.VMEM((2, RB * GK * B), jnp.float32),
            pltpu.VMEM((B, DP // 2), jnp.int32),
            pltpu.VMEM((B, DP // 2), jnp.int32),
            pltpu.VMEM((B, D), jnp.float32),
            pltpu.VMEM_SHARED((N, D), jnp.float32),
            pltpu.SemaphoreType.DMA,
            pltpu.SemaphoreType.DMA,
            pltpu.SemaphoreType.DMA,
        ],
    )
    def sc_spmm(p_hbm, rec_hbm, w_hbm, out_hbm,
                recs_v, wsl_v, rows0_v, rows1_v, msg_v, acc_sh,
                sem0, sem1, rsem):
        cid = lax.axis_index("c")
        sid = lax.axis_index("s")
        wid = sid * NC + cid
        rows_v = (rows0_v, rows1_v)
        sem = (sem0, sem1)

        def zero_row(r, carry):
            for dd in range(D // L):
                msg_v[r, pl.ds(dd * L, L)] = jnp.zeros((L,), jnp.float32)
            return carry
        lax.fori_loop(0, WB, zero_row, 0)
        row0 = sid * SB
        for b in range(NBLK):
            @pl.when(row0 + b * WB < N)
            def _():
                pltpu.sync_copy(msg_v.at[pl.ds(0, WB)],
                                acc_sh.at[pl.ds(row0 + b * WB, WB)])
        plsc.subcore_barrier()

        is_a = wid < XA
        g0 = jnp.where(is_a, wid * CA, XA * CA + (wid - XA) * CB)
        gs0 = jnp.where(is_a, wid * (CA // RB),
                        XA * (CA // RB) + (wid - XA) * (CB // RB))
        nslab = jnp.where(is_a, CA // RB, CB // RB)

        def copy_slab(ss, sb):
            pltpu.async_copy(rec_hbm.at[pl.ds(g0 + ss * RB, RB)],
                             recs_v.at[sb], rsem)
            pltpu.async_copy(w_hbm.at[gs0 + ss], wsl_v.at[sb], rsem)

        def wait_slab(ss, sb):
            pltpu.make_async_copy(rec_hbm.at[pl.ds(g0 + ss * RB, RB)],
                                  recs_v.at[sb], rsem).wait()
            pltpu.make_async_copy(w_hbm.at[gs0 + ss], wsl_v.at[sb],
                                  rsem).wait()

        def gather(sb, q, s):
            pltpu.async_copy(p_hbm.at[recs_v.at[sb, q, 0]], rows_v[s],
                             sem[s])

        def wait_gather(sb, q, s):
            pltpu.make_async_copy(p_hbm.at[recs_v.at[sb, q, 0]], rows_v[s],
                                  sem[s]).wait()

        def compute_scatter(sb, q, s):
            rv = rows_v[s]

            def _edges(j, carry2):
                wvec = [
                    plsc.load_gather(
                        wsl_v,
                        [jnp.full((L,), sb, jnp.int32),
                         jnp.full((L,), (q * GK + c) * B + j, jnp.int32)])
                    for c in range(GK)
                ]
                for dd in range(D // (2 * L)):
                    ma = None
                    mb = None
                    for c in range(GK):
                        words = rv[j, pl.ds(c * (D // 2) + dd * L, L)]
                        ab = plsc.bitcast(words, jnp.bfloat16)
                        a, b = plsc.unpack(
                            ab, format=plsc.PackFormat.INTERLEAVED,
                            preferred_element_type=jnp.float32)
                        ma = (wvec[c] * a if ma is None
                              else ma + wvec[c] * a)
                        mb = (wvec[c] * b if mb is None
                              else mb + wvec[c] * b)
                    msg_v[j, pl.ds(dd * 2 * L, L)] = ma
                    msg_v[j, pl.ds(dd * 2 * L + L, L)] = mb
                return carry2
            lax.fori_loop(0, B, _edges, 0)

            pltpu.sync_copy(msg_v, acc_sh.at[recs_v.at[sb, q, 1]], add=True)

        copy_slab(0, 0)
        wait_slab(0, 0)
        copy_slab(1, 1)
        gather(0, 0, 0)

        def slab_body(ss, carry):
            sb = (ss % 2).astype(jnp.int32)
            for q in range(RB):
                s = q % 2
                if q < RB - 1:
                    gather(sb, q + 1, 1 - s)
                else:
                    @pl.when(ss + 1 < nslab)
                    def _():
                        wait_slab(ss + 1, 1 - sb)
                        gather(1 - sb, 0, 1 - s)
                wait_gather(sb, q, s)
                compute_scatter(sb, q, s)
            @pl.when(ss + 2 < nslab)
            def _():
                copy_slab(ss + 2, sb)
            return carry
        lax.fori_loop(0, nslab, slab_body, 0)

        plsc.subcore_barrier()
        for b in range(NBLK):
            @pl.when(row0 + b * WB < N)
            def _():
                r0 = row0 + b * WB
                pltpu.sync_copy(acc_sh.at[pl.ds(r0, WB)],
                                msg_v.at[pl.ds(0, WB)])
                pltpu.sync_copy(msg_v.at[pl.ds(0, WB)],
                                out_hbm.at[cid, pl.ds(r0, WB)])

    return sc_spmm, NC, B, RB


def kernel(h, x, edge_weight, weights, inp_weights, graph_mixing_weight,
           inp_graph_mixing_weight, edge_index):
    N, D_IN = h.shape
    D_NET = x.shape[1]
    G, K, _, D_OUT = weights.shape
    E = edge_index.shape[1]
    GK = G * K
    DP = GK * D_OUT

    wh = weights * graph_mixing_weight[:, 0][:, None, None, None]
    wx = inp_weights * inp_graph_mixing_weight[:, 0][:, None, None, None]
    wh = wh.reshape(GK, D_IN, D_OUT).transpose(1, 0, 2).reshape(D_IN, DP)
    wx = wx.reshape(GK, D_NET, D_OUT).transpose(1, 0, 2).reshape(D_NET, DP)
    wcat = jnp.concatenate([wh, wx], axis=0)
    m = jnp.arange(D_OUT)
    jcol = 16 * (2 * (m // 32) + (m % 32) % 2) + (m % 32) // 2
    colperm = jnp.concatenate([c * D_OUT + jcol for c in range(GK)])
    wcat = wcat[:, colperm]
    hx = jnp.concatenate([h, x], axis=1)

    BLK = 1000
    p = pl.pallas_call(
        _matmul_body,
        grid=(N // BLK,),
        in_specs=[
            pl.BlockSpec((BLK, D_IN + D_NET), lambda i: (i, 0)),
            pl.BlockSpec((D_IN + D_NET, DP), lambda i: (0, 0)),
        ],
        out_specs=pl.BlockSpec((BLK, DP), lambda i: (i, 0)),
        out_shape=jax.ShapeDtypeStruct((N, DP), jnp.bfloat16),
    )(hx, wcat)

    sc_spmm, NC, B, RB = _make_sc_spmm(N, E, GK, D_OUT, DP)
    TT = E // B
    eint = edge_index.reshape(2, TT, B).transpose(1, 0, 2)
    wpack = (edge_weight.reshape(GK, TT, B).transpose(1, 0, 2)
             .reshape(TT // RB, RB * GK * B))
    p32 = jax.lax.bitcast_convert_type(
        p.reshape(N, DP // 2, 2), jnp.int32)
    parts = sc_spmm(p32, eint, wpack)

    out = pl.pallas_call(
        _addrelu_body,
        grid=(N // BLK,),
        in_specs=[
            pl.BlockSpec((BLK, D_OUT), lambda i: (i, 0)),
            pl.BlockSpec((BLK, D_OUT), lambda i: (i, 0)),
        ],
        out_specs=pl.BlockSpec((BLK, D_OUT), lambda i: (i, 0)),
        out_shape=jax.ShapeDtypeStruct((N, D_OUT), jnp.float32),
    )(parts[0], parts[1])
    return out
